# Initial kernel scaffold; baseline (speedup 1.0000x reference)
import jax
import jax.numpy as jnp
from jax.experimental import pallas as pl


def kernel(inputs, codebook):
    raise NotImplementedError("write your pallas kernel here")


# speedup vs baseline: 3.6686x; 3.6686x over previous
import jax
import jax.numpy as jnp
from jax.experimental import pallas as pl
from jax.experimental.pallas import tpu as pltpu

N_EMB = 1024
EMB_DIM = 64
COMMITMENT_COST = 0.25


def _vq_body(x_ref, cb_ref, oh_ref, q_ref, loss_ref, perp_ref,
             acc_loss, acc_counts):
    b = pl.program_id(0)
    nb = pl.num_programs(0)
    xT = x_ref[0]
    cb = cb_ref[...]

    sx = jnp.sum(xT * xT, axis=0)
    scb = jnp.sum(cb * cb, axis=1)
    m = jax.lax.dot_general(cb, xT, (((1,), (0,)), ((), ())),
                            preferred_element_type=jnp.float32)
    dist = (sx[None, :] + scb[:, None]) - 2.0 * m
    idx = jnp.argmin(dist, axis=0)

    eiota = jax.lax.broadcasted_iota(jnp.int32, dist.shape, 0)
    ohT = (eiota == idx[None, :]).astype(jnp.float32)
    oh_ref[0] = ohT

    qT = jax.lax.dot_general(cb, ohT, (((0,), (0,)), ((), ())),
                             preferred_element_type=jnp.float32)
    q_ref[0] = qT

    part_loss = jnp.sum((qT - xT) ** 2).reshape(1, 1)
    part_counts = jnp.sum(ohT, axis=1)

    @pl.when(b == 0)
    def _init():
        acc_loss[...] = part_loss
        acc_counts[...] = part_counts

    @pl.when(b > 0)
    def _acc():
        acc_loss[...] += part_loss
        acc_counts[...] += part_counts

    @pl.when(b == nb - 1)
    def _finish():
        n_tok = jnp.float32(nb * xT.shape[1])
        loss_ref[...] = (COMMITMENT_COST / (n_tok * EMB_DIM)) * acc_loss[...]
        avg = acc_counts[...] / n_tok
        perp_ref[...] = jnp.exp(-jnp.sum(avg * jnp.log(avg + 1e-10))).reshape(1, 1)


def kernel(inputs, codebook):
    B, C, H, W = inputs.shape
    HW = H * W
    x3 = inputs.reshape(B, C, HW)

    oh, q3, loss2, perp2 = pl.pallas_call(
        _vq_body,
        grid=(B,),
        in_specs=[
            pl.BlockSpec((1, C, HW), lambda b: (b, 0, 0)),
            pl.BlockSpec((N_EMB, EMB_DIM), lambda b: (0, 0)),
        ],
        out_specs=[
            pl.BlockSpec((1, N_EMB, HW), lambda b: (b, 0, 0)),
            pl.BlockSpec((1, C, HW), lambda b: (b, 0, 0)),
            pl.BlockSpec((1, 1), lambda b: (0, 0)),
            pl.BlockSpec((1, 1), lambda b: (0, 0)),
        ],
        out_shape=[
            jax.ShapeDtypeStruct((B, N_EMB, HW), jnp.float32),
            jax.ShapeDtypeStruct((B, C, HW), jnp.float32),
            jax.ShapeDtypeStruct((1, 1), jnp.float32),
            jax.ShapeDtypeStruct((1, 1), jnp.float32),
        ],
        scratch_shapes=[
            pltpu.VMEM((1, 1), jnp.float32),
            pltpu.VMEM((N_EMB,), jnp.float32),
        ],
        compiler_params=pltpu.CompilerParams(
            dimension_semantics=("arbitrary",),
        ),
    )(x3, codebook)

    loss = loss2[0, 0]
    perplexity = perp2[0, 0]
    quantized_st = q3.reshape(B, C, H, W)
    return loss, quantized_st, perplexity, oh


# speedup vs baseline: 3.8398x; 1.0467x over previous
import jax
import jax.numpy as jnp
from jax.experimental import pallas as pl
from jax.experimental.pallas import tpu as pltpu

N_EMB = 1024
EMB_DIM = 64
COMMITMENT_COST = 0.25


def _vq_body(x_ref, cb_ref, oh_ref, q_ref, loss_ref, perp_ref,
             acc_loss, acc_counts):
    b = pl.program_id(0)
    nb = pl.num_programs(0)
    xT = x_ref[0]
    cb = cb_ref[...]

    sx = jnp.sum(xT * xT, axis=0)
    scb = jnp.sum(cb * cb, axis=1)
    m = jax.lax.dot_general(cb, xT, (((1,), (0,)), ((), ())),
                            preferred_element_type=jnp.float32)
    dist = (sx[None, :] + scb[:, None]) - 2.0 * m
    idx = jnp.argmin(dist, axis=0)

    eiota = jax.lax.broadcasted_iota(jnp.int32, dist.shape, 0)
    ohT = (eiota == idx[None, :]).astype(jnp.float32)
    oh_ref[0] = ohT

    qT = jax.lax.dot_general(cb, ohT, (((0,), (0,)), ((), ())),
                             preferred_element_type=jnp.float32)
    q_ref[0] = qT

    part_loss = jnp.sum((qT - xT) ** 2).reshape(1, 1)
    ones_n = jnp.ones((ohT.shape[1], 8), jnp.float32)
    part_counts = jax.lax.dot_general(ohT, ones_n, (((1,), (0,)), ((), ())),
                                      preferred_element_type=jnp.float32)

    @pl.when(b == 0)
    def _init():
        acc_loss[...] = part_loss
        acc_counts[...] = part_counts

    @pl.when(b > 0)
    def _acc():
        acc_loss[...] += part_loss
        acc_counts[...] += part_counts

    @pl.when(b == nb - 1)
    def _finish():
        n_tok = jnp.float32(nb * xT.shape[1])
        loss_ref[...] = (COMMITMENT_COST / (n_tok * EMB_DIM)) * acc_loss[...]
        avg = acc_counts[...] / n_tok
        ent = jnp.sum(avg * jnp.log(avg + 1e-10)) / 8.0
        perp_ref[...] = jnp.exp(-ent).reshape(1, 1)


def kernel(inputs, codebook):
    B, C, H, W = inputs.shape
    HW = H * W
    x3 = inputs.reshape(B, C, HW)

    oh, q3, loss2, perp2 = pl.pallas_call(
        _vq_body,
        grid=(B,),
        in_specs=[
            pl.BlockSpec((1, C, HW), lambda b: (b, 0, 0)),
            pl.BlockSpec((N_EMB, EMB_DIM), lambda b: (0, 0)),
        ],
        out_specs=[
            pl.BlockSpec((1, N_EMB, HW), lambda b: (b, 0, 0)),
            pl.BlockSpec((1, C, HW), lambda b: (b, 0, 0)),
            pl.BlockSpec((1, 1), lambda b: (0, 0)),
            pl.BlockSpec((1, 1), lambda b: (0, 0)),
        ],
        out_shape=[
            jax.ShapeDtypeStruct((B, N_EMB, HW), jnp.float32),
            jax.ShapeDtypeStruct((B, C, HW), jnp.float32),
            jax.ShapeDtypeStruct((1, 1), jnp.float32),
            jax.ShapeDtypeStruct((1, 1), jnp.float32),
        ],
        scratch_shapes=[
            pltpu.VMEM((1, 1), jnp.float32),
            pltpu.VMEM((N_EMB, 8), jnp.float32),
        ],
        compiler_params=pltpu.CompilerParams(
            dimension_semantics=("arbitrary",),
        ),
    )(x3, codebook)

    loss = loss2[0, 0]
    perplexity = perp2[0, 0]
    quantized_st = q3.reshape(B, C, H, W)
    return loss, quantized_st, perplexity, oh
